# baseline (device time: 12590 ns/iter reference)
import jax
import jax.numpy as jnp
from jax import lax
from jax.experimental import pallas as pl
from jax.experimental.pallas import tpu as pltpu

CHUNKS = (128, 64, 32, 16, 16)
OFFS = (0, 128, 192, 224, 240)
K = len(CHUNKS)


def kernel(x):
    m, n = x.shape
    half = m // 2

    def body(
        x_hbm,
        out_ref,
        xv,
        send1,
        recv1,
        recv2,
        dma_sem,
        send_sem1,
        recv_sem1,
        send_sem2,
        recv_sem2,
    ):
        my_x = lax.axis_index("x")
        my_y = lax.axis_index("y")
        y_nbr = (my_x, 1 - my_y)
        x_nbr = (1 - my_x, my_y)
        my_base = my_x * half
        other_base = (1 - my_x) * half

        ldma = pltpu.make_async_copy(x_hbm, xv, dma_sem)
        ldma.start()

        barrier = pltpu.get_barrier_semaphore()
        for nbr in (y_nbr, x_nbr):
            pl.semaphore_signal(
                barrier, inc=1, device_id=nbr,
                device_id_type=pl.DeviceIdType.MESH,
            )
        pl.semaphore_wait(barrier, 2)
        ldma.wait()

        def rdma1(c):
            return pltpu.make_async_remote_copy(
                src_ref=send1.at[pl.ds(OFFS[c], CHUNKS[c])],
                dst_ref=recv1.at[pl.ds(OFFS[c], CHUNKS[c])],
                send_sem=send_sem1.at[c],
                recv_sem=recv_sem1.at[c],
                device_id=y_nbr,
                device_id_type=pl.DeviceIdType.MESH,
            )

        def rdma2(c):
            return pltpu.make_async_remote_copy(
                src_ref=recv1.at[pl.ds(OFFS[c], CHUNKS[c])],
                dst_ref=recv2.at[pl.ds(OFFS[c], CHUNKS[c])],
                send_sem=send_sem2.at[c],
                recv_sem=recv_sem2.at[c],
                device_id=x_nbr,
                device_id_type=pl.DeviceIdType.MESH,
            )

        for c in range(K):
            send1[pl.ds(OFFS[c], CHUNKS[c]), :] = xv[
                pl.ds(my_base + OFFS[c], CHUNKS[c]), :
            ].astype(jnp.bfloat16)
            rdma1(c).start()

        for c in range(K):
            rdma1(c).wait_recv()
            rdma2(c).start()
            out_ref[pl.ds(my_base + OFFS[c], CHUNKS[c]), :] = (
                send1[pl.ds(OFFS[c], CHUNKS[c]), :]
                + recv1[pl.ds(OFFS[c], CHUNKS[c]), :]
            )

        for c in range(K):
            rdma2(c).wait_recv()
            out_ref[pl.ds(other_base + OFFS[c], CHUNKS[c]), :] = (
                xv[pl.ds(other_base + OFFS[c], CHUNKS[c]), :]
                + recv2[pl.ds(OFFS[c], CHUNKS[c]), :].astype(jnp.float32)
            ).astype(jnp.bfloat16)

        for c in range(K):
            rdma1(c).wait_send()
            rdma2(c).wait_send()

    return pl.pallas_call(
        body,
        out_shape=jax.ShapeDtypeStruct((m, n), jnp.bfloat16),
        in_specs=[pl.BlockSpec(memory_space=pl.ANY)],
        out_specs=pl.BlockSpec(memory_space=pltpu.VMEM),
        scratch_shapes=[
            pltpu.VMEM((m, n), jnp.float32),
            pltpu.VMEM((half, n), jnp.bfloat16),
            pltpu.VMEM((half, n), jnp.bfloat16),
            pltpu.VMEM((half, n), jnp.bfloat16),
            pltpu.SemaphoreType.DMA,
            pltpu.SemaphoreType.DMA((K,)),
            pltpu.SemaphoreType.DMA((K,)),
            pltpu.SemaphoreType.DMA((K,)),
            pltpu.SemaphoreType.DMA((K,)),
        ],
        compiler_params=pltpu.CompilerParams(collective_id=0),
    )(x)


# device time: 12473 ns/iter; 1.0094x vs baseline; 1.0094x over previous
import jax
import jax.numpy as jnp
from jax import lax
from jax.experimental import pallas as pl
from jax.experimental.pallas import tpu as pltpu

CHUNKS = (128, 64, 32, 16, 16)
OFFS = (0, 128, 192, 224, 240)
K = len(CHUNKS)


def kernel(x):
    m, n = x.shape
    half = m // 2

    def body(
        x_ref,
        out_ref,
        send1,
        recv1,
        recv2,
        ob,
        send_sem1,
        recv_sem1,
        send_sem2,
        recv_sem2,
    ):
        my_x = lax.axis_index("x")
        my_y = lax.axis_index("y")
        y_nbr = (my_x, 1 - my_y)
        x_nbr = (1 - my_x, my_y)
        my_base = my_x * half
        other_base = (1 - my_x) * half

        barrier = pltpu.get_barrier_semaphore()
        for nbr in (y_nbr, x_nbr):
            pl.semaphore_signal(
                barrier, inc=1, device_id=nbr,
                device_id_type=pl.DeviceIdType.MESH,
            )
        pl.semaphore_wait(barrier, 2)

        def rdma1(c):
            return pltpu.make_async_remote_copy(
                src_ref=send1.at[pl.ds(OFFS[c], CHUNKS[c])],
                dst_ref=recv1.at[pl.ds(OFFS[c], CHUNKS[c])],
                send_sem=send_sem1.at[c],
                recv_sem=recv_sem1.at[c],
                device_id=y_nbr,
                device_id_type=pl.DeviceIdType.MESH,
            )

        def rdma2(c):
            return pltpu.make_async_remote_copy(
                src_ref=recv1.at[pl.ds(OFFS[c], CHUNKS[c])],
                dst_ref=recv2.at[pl.ds(OFFS[c], CHUNKS[c])],
                send_sem=send_sem2.at[c],
                recv_sem=recv_sem2.at[c],
                device_id=x_nbr,
                device_id_type=pl.DeviceIdType.MESH,
            )

        for c in range(K):
            send1[pl.ds(OFFS[c], CHUNKS[c]), :] = x_ref[
                pl.ds(my_base + OFFS[c], CHUNKS[c]), :
            ].astype(jnp.bfloat16)
            rdma1(c).start()

        ob[...] = x_ref[pl.ds(other_base, half), :].astype(jnp.bfloat16)

        for c in range(K):
            rdma1(c).wait_recv()
            rdma2(c).start()
            out_ref[pl.ds(my_base + OFFS[c], CHUNKS[c]), :] = (
                send1[pl.ds(OFFS[c], CHUNKS[c]), :]
                + recv1[pl.ds(OFFS[c], CHUNKS[c]), :]
            )

        for c in range(K):
            rdma2(c).wait_recv()
            out_ref[pl.ds(other_base + OFFS[c], CHUNKS[c]), :] = (
                ob[pl.ds(OFFS[c], CHUNKS[c]), :]
                + recv2[pl.ds(OFFS[c], CHUNKS[c]), :]
            )

        for c in range(K):
            rdma1(c).wait_send()
            rdma2(c).wait_send()

    return pl.pallas_call(
        body,
        out_shape=jax.ShapeDtypeStruct((m, n), jnp.bfloat16),
        in_specs=[pl.BlockSpec(memory_space=pltpu.VMEM)],
        out_specs=pl.BlockSpec(memory_space=pltpu.VMEM),
        scratch_shapes=[
            pltpu.VMEM((half, n), jnp.bfloat16),
            pltpu.VMEM((half, n), jnp.bfloat16),
            pltpu.VMEM((half, n), jnp.bfloat16),
            pltpu.VMEM((half, n), jnp.bfloat16),
            pltpu.SemaphoreType.DMA((K,)),
            pltpu.SemaphoreType.DMA((K,)),
            pltpu.SemaphoreType.DMA((K,)),
            pltpu.SemaphoreType.DMA((K,)),
        ],
        compiler_params=pltpu.CompilerParams(collective_id=0),
    )(x)


# device time: 11570 ns/iter; 1.0882x vs baseline; 1.0780x over previous
import jax
import jax.numpy as jnp
from jax import lax
from jax.experimental import pallas as pl
from jax.experimental.pallas import tpu as pltpu

K = 8
CHUNKS = (32,) * K
OFFS = tuple(32 * c for c in range(K))


def kernel(x):
    m, n = x.shape
    half = m // 2

    def body(
        x_ref,
        out_ref,
        send1,
        recv1,
        recv2,
        ob,
        send_sem1,
        recv_sem1,
        send_sem2,
        recv_sem2,
    ):
        my_x = lax.axis_index("x")
        my_y = lax.axis_index("y")
        y_nbr = (my_x, 1 - my_y)
        x_nbr = (1 - my_x, my_y)
        my_base = my_x * half
        other_base = (1 - my_x) * half

        barrier = pltpu.get_barrier_semaphore()
        for nbr in (y_nbr, x_nbr):
            pl.semaphore_signal(
                barrier, inc=1, device_id=nbr,
                device_id_type=pl.DeviceIdType.MESH,
            )
        pl.semaphore_wait(barrier, 2)

        def rdma1(c):
            return pltpu.make_async_remote_copy(
                src_ref=send1.at[pl.ds(OFFS[c], CHUNKS[c])],
                dst_ref=recv1.at[pl.ds(OFFS[c], CHUNKS[c])],
                send_sem=send_sem1.at[c],
                recv_sem=recv_sem1.at[c],
                device_id=y_nbr,
                device_id_type=pl.DeviceIdType.MESH,
            )

        def rdma2(c):
            return pltpu.make_async_remote_copy(
                src_ref=recv1.at[pl.ds(OFFS[c], CHUNKS[c])],
                dst_ref=recv2.at[pl.ds(OFFS[c], CHUNKS[c])],
                send_sem=send_sem2.at[c],
                recv_sem=recv_sem2.at[c],
                device_id=x_nbr,
                device_id_type=pl.DeviceIdType.MESH,
            )

        for c in range(K):
            send1[pl.ds(OFFS[c], CHUNKS[c]), :] = x_ref[
                pl.ds(my_base + OFFS[c], CHUNKS[c]), :
            ].astype(jnp.bfloat16)
            rdma1(c).start()

        ob[...] = x_ref[pl.ds(other_base, half), :].astype(jnp.bfloat16)

        for c in range(K):
            rdma1(c).wait_recv()
            rdma2(c).start()
            out_ref[pl.ds(my_base + OFFS[c], CHUNKS[c]), :] = (
                send1[pl.ds(OFFS[c], CHUNKS[c]), :]
                + recv1[pl.ds(OFFS[c], CHUNKS[c]), :]
            )

        for c in range(K):
            rdma2(c).wait_recv()
            out_ref[pl.ds(other_base + OFFS[c], CHUNKS[c]), :] = (
                ob[pl.ds(OFFS[c], CHUNKS[c]), :]
                + recv2[pl.ds(OFFS[c], CHUNKS[c]), :]
            )

        for c in range(K):
            rdma1(c).wait_send()
            rdma2(c).wait_send()

    return pl.pallas_call(
        body,
        out_shape=jax.ShapeDtypeStruct((m, n), jnp.bfloat16),
        in_specs=[pl.BlockSpec(memory_space=pltpu.VMEM)],
        out_specs=pl.BlockSpec(memory_space=pltpu.VMEM),
        scratch_shapes=[
            pltpu.VMEM((half, n), jnp.bfloat16),
            pltpu.VMEM((half, n), jnp.bfloat16),
            pltpu.VMEM((half, n), jnp.bfloat16),
            pltpu.VMEM((half, n), jnp.bfloat16),
            pltpu.SemaphoreType.DMA((K,)),
            pltpu.SemaphoreType.DMA((K,)),
            pltpu.SemaphoreType.DMA((K,)),
            pltpu.SemaphoreType.DMA((K,)),
        ],
        compiler_params=pltpu.CompilerParams(collective_id=0),
    )(x)


# device time: 10842 ns/iter; 1.1612x vs baseline; 1.0671x over previous
import jax
import jax.numpy as jnp
from jax import lax
from jax.experimental import pallas as pl
from jax.experimental.pallas import tpu as pltpu

R = 32
E = 64
HALF = 256
F = HALF - E
K1 = (HALF + E) // R
K2 = F // R


def kernel(x):
    m, n = x.shape
    half = m // 2
    assert half == HALF

    def body(
        x_ref,
        out_ref,
        send1,
        recv1,
        recv2,
        ob,
        send_sem1,
        recv_sem1,
        send_sem2,
        recv_sem2,
    ):
        my_x = lax.axis_index("x")
        my_y = lax.axis_index("y")
        y_nbr = (my_x, 1 - my_y)
        x_nbr = (1 - my_x, my_y)
        my_base = my_x * half
        other_base = (1 - my_x) * half

        barrier = pltpu.get_barrier_semaphore()
        for nbr in (y_nbr, x_nbr):
            pl.semaphore_signal(
                barrier, inc=1, device_id=nbr,
                device_id_type=pl.DeviceIdType.MESH,
            )
        pl.semaphore_wait(barrier, 2)

        def rdma1(c):
            return pltpu.make_async_remote_copy(
                src_ref=send1.at[pl.ds(c * R, R)],
                dst_ref=recv1.at[pl.ds(c * R, R)],
                send_sem=send_sem1.at[c],
                recv_sem=recv_sem1.at[c],
                device_id=y_nbr,
                device_id_type=pl.DeviceIdType.MESH,
            )

        def rdma2(c):
            return pltpu.make_async_remote_copy(
                src_ref=recv1.at[pl.ds(c * R, R)],
                dst_ref=recv2.at[pl.ds(c * R, R)],
                send_sem=send_sem2.at[c],
                recv_sem=recv_sem2.at[c],
                device_id=x_nbr,
                device_id_type=pl.DeviceIdType.MESH,
            )

        for c in range(K1):
            if c * R < half:
                src_row = my_base + c * R
            else:
                src_row = other_base + F + (c * R - half)
            send1[pl.ds(c * R, R), :] = x_ref[pl.ds(src_row, R), :].astype(
                jnp.bfloat16
            )
            rdma1(c).start()

        ob[...] = x_ref[pl.ds(other_base, half), :].astype(jnp.bfloat16)

        def store_x(f):
            out_ref[pl.ds(other_base + f * R, R), :] = (
                ob[pl.ds(f * R, R), :] + recv2[pl.ds(f * R, R), :]
            )

        for c in range(K1):
            rdma1(c).wait_recv()
            if c < K2:
                rdma2(c).start()
            if c * R < half:
                out_ref[pl.ds(my_base + c * R, R), :] = (
                    send1[pl.ds(c * R, R), :] + recv1[pl.ds(c * R, R), :]
                )
            else:
                r = F + (c * R - half)
                out_ref[pl.ds(other_base + r, R), :] = (
                    ob[pl.ds(r, R), :] + recv1[pl.ds(c * R, R), :]
                )
            if K2 < c < K2 + 4:
                rdma2(c - K2 - 1).wait_recv()
                store_x(c - K2 - 1)

        for f in range(3, K2):
            rdma2(f).wait_recv()
            store_x(f)

        for c in range(K1):
            rdma1(c).wait_send()
        for c in range(K2):
            rdma2(c).wait_send()

    return pl.pallas_call(
        body,
        out_shape=jax.ShapeDtypeStruct((m, n), jnp.bfloat16),
        in_specs=[pl.BlockSpec(memory_space=pltpu.VMEM)],
        out_specs=pl.BlockSpec(memory_space=pltpu.VMEM),
        scratch_shapes=[
            pltpu.VMEM((half + E, n), jnp.bfloat16),
            pltpu.VMEM((half + E, n), jnp.bfloat16),
            pltpu.VMEM((F, n), jnp.bfloat16),
            pltpu.VMEM((half, n), jnp.bfloat16),
            pltpu.SemaphoreType.DMA((K1,)),
            pltpu.SemaphoreType.DMA((K1,)),
            pltpu.SemaphoreType.DMA((K2,)),
            pltpu.SemaphoreType.DMA((K2,)),
        ],
        compiler_params=pltpu.CompilerParams(collective_id=0),
    )(x)
